# baseline (speedup 1.0000x reference)
import jax
import jax.numpy as jnp
from jax import lax
from jax.experimental import pallas as pl
from jax.experimental.pallas import tpu as pltpu
from jax.experimental.pallas import tpu_sc as plsc

S = 2048
R = 128
A = 16
MEM = 64
NC, NS = 2, 16
NW = NC * NS
P_PER_W = S // NW
K_PER_W = P_PER_W * R
HIST = 2 * R
CHUNK = K_PER_W + HIST
TR = S // 8
RH = R // 2
V3 = R * A ** 3
VT3 = V3 // 128
VT_PER_W = VT3 // NW


def _tr_body(lv_hbm, t3lin_hbm, wout_v, isem, osem):
    cid = lax.axis_index("c")
    sid = lax.axis_index("s")
    wid = sid * NC + cid
    vt0 = wid * VT_PER_W

    def issue_in(i, d):
        for mb in range(8):
            for mi in range(8):
                pltpu.async_copy(lv_hbm.at[mb, vt0 + i, mi],
                                 wout_v.at[d, :, pl.ds(mb * 8 + mi, 1)], isem)

    def issue_out(i, d):
        pltpu.async_copy(wout_v.at[d],
                         t3lin_hbm.at[pl.ds((vt0 + i) * 128, 128)], osem)

    def drain_i(d):
        pltpu.make_async_copy(t3lin_hbm.at[pl.ds(0, 128)],
                              wout_v.at[d], isem).wait()

    def drain_o(d):
        pltpu.make_async_copy(t3lin_hbm.at[pl.ds(0, 128)],
                              wout_v.at[d], osem).wait()

    issue_in(0, 0)

    @pl.loop(0, VT_PER_W, step=2)
    def _(i2):
        for d in range(2):
            i = i2 + d
            drain_i(d)

            @pl.when(i >= 1)
            def _():
                drain_o(1 - d)

            @pl.when(i + 1 < VT_PER_W)
            def _():
                issue_in(i + 1, 1 - d)
            issue_out(i, d)

    drain_o(1)


def _sc_body(codes_hbm, t2_hbm, t3_hbm, out_hbm,
             codes_v, idx2_v, idx3_v, gbuf_v, zbuf_v, gsem, ssem):
    cid = lax.axis_index("c")
    sid = lax.axis_index("s")
    wid = sid * NC + cid
    base_k = wid * K_PER_W
    start = pl.multiple_of(jnp.maximum(base_k - HIST, 0), HIST)
    off = base_k - start

    pltpu.sync_copy(codes_hbm.at[pl.ds(start, CHUNK)], codes_v)

    lanes2 = lax.iota(jnp.int32, 16) * 2

    @pl.loop(0, P_PER_W)
    def _(p):
        for rb in range(2):
            for jjr in range(RH // 16):
                kvec = (off + p * R + rb + jjr * 32) + lanes2
                c0 = plsc.load_gather(codes_v, [kvec])
                c1 = plsc.load_gather(codes_v, [jnp.maximum(kvec - R, 0)])
                c2 = plsc.load_gather(codes_v, [jnp.maximum(kvec - 2 * R, 0)])
                rvec = (lanes2 + (jjr * 32 + rb))
                idx2_v[p, rb, pl.ds(jjr * 16, 16)] = (
                    rvec * (A * A) + c1 + c0 * A)
                idx3_v[p, rb, pl.ds(jjr * 16, 16)] = (
                    rvec * (A * A * A) + c2 + c1 * A + c0 * (A * A))

    @pl.when(wid == 0)
    def _():
        @pl.loop(0, MEM)
        def _(i):
            for c in range(2 * MEM // 16):
                zbuf_v[i, pl.ds(c * 16, 16)] = jnp.zeros((16,), jnp.float32)

    def issue_gathers(p, b):
        pltpu.async_copy(t2_hbm.at[idx2_v.at[p, 0]], gbuf_v.at[b, 0, 0], gsem)
        pltpu.async_copy(t2_hbm.at[idx2_v.at[p, 1]], gbuf_v.at[b, 0, 1], gsem)
        pltpu.async_copy(t3_hbm.at[idx3_v.at[p, 0]], gbuf_v.at[b, 1, 0], gsem)
        pltpu.async_copy(t3_hbm.at[idx3_v.at[p, 1]], gbuf_v.at[b, 1, 1], gsem)

    def drain(sem, tile_r, srem, b):
        pltpu.make_async_copy(out_hbm.at[tile_r, :, :, srem, pl.ds(0, MEM)],
                              gbuf_v.at[b, 0], sem).wait()
        pltpu.make_async_copy(out_hbm.at[tile_r, :, :, srem, pl.ds(0, MEM)],
                              gbuf_v.at[b, 1], sem).wait()

    issue_gathers(0, 0)
    issue_gathers(1, 1)

    @pl.loop(0, P_PER_W, step=2)
    def _(p):
        for b in range(2):
            q = p + b
            tile_r = wid * (P_PER_W // 8) + q // 8
            srem = q % 8
            drain(gsem, tile_r, srem, b)
            for o in range(2):
                for rb in range(2):
                    pltpu.async_copy(
                        gbuf_v.at[b, o, rb],
                        out_hbm.at[tile_r, o, :, srem, pl.ds(rb * MEM, MEM)],
                        ssem)
            drain(ssem, tile_r, srem, b)

            @pl.when(q + 2 < P_PER_W)
            def _():
                issue_gathers(q + 2, b)

    @pl.when(wid == 0)
    def _():
        pltpu.sync_copy(zbuf_v, out_hbm.at[0, 0, :, 0, :])
        pltpu.sync_copy(zbuf_v, out_hbm.at[0, 1, :, 0, :])
        pltpu.sync_copy(zbuf_v, out_hbm.at[0, 1, :, 1, :])


_mesh = plsc.VectorSubcoreMesh(core_axis_name="c", subcore_axis_name="s",
                               num_cores=NC, num_subcores=NS)

_tr_call = pl.kernel(
    _tr_body,
    out_type=jax.ShapeDtypeStruct((V3, MEM), jnp.float32),
    mesh=_mesh,
    scratch_types=[
        pltpu.VMEM((2, 128, MEM), jnp.float32),
        pltpu.SemaphoreType.DMA,
        pltpu.SemaphoreType.DMA,
    ],
    compiler_params=pltpu.CompilerParams(use_tc_tiling_on_sc=False,
                                         needs_layout_passes=False),
)

_sc_call = pl.kernel(
    _sc_body,
    out_type=jax.ShapeDtypeStruct((TR, 2, MEM, 8, 2 * MEM), jnp.float32),
    mesh=_mesh,
    scratch_types=[
        pltpu.VMEM((CHUNK,), jnp.int32),
        pltpu.VMEM((P_PER_W, 2, RH), jnp.int32),
        pltpu.VMEM((P_PER_W, 2, RH), jnp.int32),
        pltpu.VMEM((2, 2, 2, RH, MEM), jnp.float32),
        pltpu.VMEM((MEM, 2 * MEM), jnp.float32),
        pltpu.SemaphoreType.DMA,
        pltpu.SemaphoreType.DMA,
    ],
    compiler_params=pltpu.CompilerParams(use_tc_tiling_on_sc=False,
                                         needs_layout_passes=False),
)


@jax.jit
def _run(codes_flat, t2, lv3):
    t3lin = _tr_call(lv3)
    return _sc_call(codes_flat, t2, t3lin)


def kernel(route_codes_btr, table_ngram_2, table_ngram_3):
    b, s, r = route_codes_btr.shape
    assert (b, s, r) == (1, S, R)
    codes_flat = route_codes_btr.astype(jnp.int32).reshape(-1)
    lv3 = (table_ngram_3.T.reshape(8, 8, VT3, 128)
           .transpose(0, 2, 1, 3).reshape(8, VT3, 8, 128, 1))
    out5 = _run(codes_flat, table_ngram_2, lv3)
    return out5.transpose(0, 3, 1, 2, 4).reshape(1, S, 2 * R * MEM)


# speedup vs baseline: 137.4211x; 137.4211x over previous
import jax
import jax.numpy as jnp
from jax import lax
from jax.experimental import pallas as pl
from jax.experimental.pallas import tpu as pltpu
from jax.experimental.pallas import tpu_sc as plsc

S = 2048
R = 128
A = 16
MEM = 64
NC, NS = 2, 16
NW = NC * NS
P_PER_W = S // NW
K_PER_W = P_PER_W * R
HIST = 2 * R
CHUNK = K_PER_W + HIST
TR = S // 8
RH = R // 2


def _sc_body(codes_hbm, t2_hbm, t3_hbm, out_hbm,
             codes_v, idx2_v, idx3_v, gbuf_v, zbuf_v, gsem, ssem):
    cid = lax.axis_index("c")
    sid = lax.axis_index("s")
    wid = sid * NC + cid
    base_k = wid * K_PER_W
    start = pl.multiple_of(jnp.maximum(base_k - HIST, 0), HIST)
    off = base_k - start

    pltpu.sync_copy(codes_hbm.at[pl.ds(start, CHUNK)], codes_v)

    lanes2 = lax.iota(jnp.int32, 16) * 2

    @pl.loop(0, P_PER_W)
    def _(p):
        for rb in range(2):
            for jjr in range(RH // 16):
                kvec = (off + p * R + rb + jjr * 32) + lanes2
                c0 = plsc.load_gather(codes_v, [kvec])
                c1 = plsc.load_gather(codes_v, [jnp.maximum(kvec - R, 0)])
                c2 = plsc.load_gather(codes_v, [jnp.maximum(kvec - 2 * R, 0)])
                rvec = (lanes2 + (jjr * 32 + rb))
                idx2_v[p, rb, pl.ds(jjr * 16, 16)] = (
                    rvec * (A * A) + c1 + c0 * A)
                idx3_v[p, rb, pl.ds(jjr * 16, 16)] = (
                    rvec * (A * A * A) + c2 + c1 * A + c0 * (A * A))

    @pl.when(wid == 0)
    def _():
        @pl.loop(0, MEM)
        def _(i):
            for c in range(2 * MEM // 16):
                zbuf_v[i, pl.ds(c * 16, 16)] = jnp.zeros((16,), jnp.float32)

    def issue_gathers(p, b):
        pltpu.async_copy(t2_hbm.at[idx2_v.at[p, 0]], gbuf_v.at[b, 0, 0], gsem)
        pltpu.async_copy(t2_hbm.at[idx2_v.at[p, 1]], gbuf_v.at[b, 0, 1], gsem)
        pltpu.async_copy(t3_hbm.at[idx3_v.at[p, 0]], gbuf_v.at[b, 1, 0], gsem)
        pltpu.async_copy(t3_hbm.at[idx3_v.at[p, 1]], gbuf_v.at[b, 1, 1], gsem)

    def drain(sem, tile_r, srem, b):
        pltpu.make_async_copy(out_hbm.at[tile_r, :, :, srem, pl.ds(0, MEM)],
                              gbuf_v.at[b, 0], sem).wait()
        pltpu.make_async_copy(out_hbm.at[tile_r, :, :, srem, pl.ds(0, MEM)],
                              gbuf_v.at[b, 1], sem).wait()

    issue_gathers(0, 0)
    issue_gathers(1, 1)

    @pl.loop(0, P_PER_W, step=2)
    def _(p):
        for b in range(2):
            q = p + b
            tile_r = wid * (P_PER_W // 8) + q // 8
            srem = q % 8
            drain(gsem, tile_r, srem, b)
            for o in range(2):
                for rb in range(2):
                    pltpu.async_copy(
                        gbuf_v.at[b, o, rb],
                        out_hbm.at[tile_r, o, :, srem, pl.ds(rb * MEM, MEM)],
                        ssem)
            drain(ssem, tile_r, srem, b)

            @pl.when(q + 2 < P_PER_W)
            def _():
                issue_gathers(q + 2, b)

    @pl.when(wid == 0)
    def _():
        pltpu.sync_copy(zbuf_v, out_hbm.at[0, 0, :, 0, :])
        pltpu.sync_copy(zbuf_v, out_hbm.at[0, 1, :, 0, :])
        pltpu.sync_copy(zbuf_v, out_hbm.at[0, 1, :, 1, :])


_mesh = plsc.VectorSubcoreMesh(core_axis_name="c", subcore_axis_name="s",
                               num_cores=NC, num_subcores=NS)

_sc_call = pl.kernel(
    _sc_body,
    out_type=jax.ShapeDtypeStruct((TR, 2, MEM, 8, 2 * MEM), jnp.float32),
    mesh=_mesh,
    scratch_types=[
        pltpu.VMEM((CHUNK,), jnp.int32),
        pltpu.VMEM((P_PER_W, 2, RH), jnp.int32),
        pltpu.VMEM((P_PER_W, 2, RH), jnp.int32),
        pltpu.VMEM((2, 2, 2, RH, MEM), jnp.float32),
        pltpu.VMEM((MEM, 2 * MEM), jnp.float32),
        pltpu.SemaphoreType.DMA,
        pltpu.SemaphoreType.DMA,
    ],
    compiler_params=pltpu.CompilerParams(use_tc_tiling_on_sc=False,
                                         needs_layout_passes=False),
)


@jax.jit
def _run(codes_flat, t2, t3):
    return _sc_call(codes_flat, t2, t3)


def kernel(route_codes_btr, table_ngram_2, table_ngram_3):
    b, s, r = route_codes_btr.shape
    assert (b, s, r) == (1, S, R)
    codes_flat = route_codes_btr.astype(jnp.int32).reshape(-1)
    out5 = _run(codes_flat, table_ngram_2, table_ngram_3)
    return out5.transpose(0, 3, 1, 2, 4).reshape(1, S, 2 * R * MEM)


# speedup vs baseline: 138.0649x; 1.0047x over previous
import jax
import jax.numpy as jnp
from jax import lax
from jax.experimental import pallas as pl
from jax.experimental.pallas import tpu as pltpu
from jax.experimental.pallas import tpu_sc as plsc

S = 2048
R = 128
A = 16
MEM = 64
NC, NS = 2, 16
NW = NC * NS
P_PER_W = S // NW
K_PER_W = P_PER_W * R
HIST = 2 * R
CHUNK = K_PER_W + HIST
TR = S // 8
RH = R // 2


def _sc_body(codes_hbm, t2_hbm, t3_hbm, out_hbm,
             codes_v, idx2_v, idx3_v, gbuf_v, zbuf_v, gsem, ssem):
    cid = lax.axis_index("c")
    sid = lax.axis_index("s")
    wid = sid * NC + cid
    base_k = wid * K_PER_W
    start = pl.multiple_of(jnp.maximum(base_k - HIST, 0), HIST)
    off = base_k - start

    pltpu.sync_copy(codes_hbm.at[pl.ds(start, CHUNK)], codes_v)

    lanes2 = lax.iota(jnp.int32, 16) * 2

    @pl.loop(0, P_PER_W)
    def _(p):
        for rb in range(2):
            for jjr in range(RH // 16):
                kvec = (off + p * R + rb + jjr * 32) + lanes2
                c0 = plsc.load_gather(codes_v, [kvec])
                c1 = plsc.load_gather(codes_v, [jnp.maximum(kvec - R, 0)])
                c2 = plsc.load_gather(codes_v, [jnp.maximum(kvec - 2 * R, 0)])
                rvec = (lanes2 + (jjr * 32 + rb))
                idx2_v[p, rb, pl.ds(jjr * 16, 16)] = (
                    rvec * (A * A) + c1 + c0 * A)
                idx3_v[p, rb, pl.ds(jjr * 16, 16)] = (
                    rvec * (A * A * A) + c2 + c1 * A + c0 * (A * A))

    @pl.when(wid == 0)
    def _():
        @pl.loop(0, MEM)
        def _(i):
            for c in range(2 * MEM // 16):
                zbuf_v[i, pl.ds(c * 16, 16)] = jnp.zeros((16,), jnp.float32)

    def issue_gathers(p, b):
        pltpu.async_copy(t2_hbm.at[idx2_v.at[p, 0]], gbuf_v.at[b, 0, 0], gsem)
        pltpu.async_copy(t2_hbm.at[idx2_v.at[p, 1]], gbuf_v.at[b, 0, 1], gsem)
        pltpu.async_copy(t3_hbm.at[idx3_v.at[p, 0]], gbuf_v.at[b, 1, 0], gsem)
        pltpu.async_copy(t3_hbm.at[idx3_v.at[p, 1]], gbuf_v.at[b, 1, 1], gsem)

    def drain(sem, tile_r, srem, b):
        pltpu.make_async_copy(out_hbm.at[tile_r, :, :, srem, pl.ds(0, MEM)],
                              gbuf_v.at[b, 0], sem).wait()
        pltpu.make_async_copy(out_hbm.at[tile_r, :, :, srem, pl.ds(0, MEM)],
                              gbuf_v.at[b, 1], sem).wait()

    issue_gathers(0, 0)
    issue_gathers(1, 1)

    @pl.loop(0, P_PER_W, step=4)
    def _(p):
        for b in range(4):
            q = p + b
            tile_r = wid * (P_PER_W // 8) + q // 8
            srem = q % 8
            drain(gsem, tile_r, srem, b)
            for o in range(2):
                for rb in range(2):
                    pltpu.async_copy(
                        gbuf_v.at[b, o, rb],
                        out_hbm.at[tile_r, o, :, srem, pl.ds(rb * MEM, MEM)],
                        ssem)

            @pl.when(q >= 2)
            def _():
                drain(ssem, tile_r, srem, b)

            @pl.when(q + 2 < P_PER_W)
            def _():
                issue_gathers(q + 2, (b + 2) % 4)

    tile_l = wid * (P_PER_W // 8) + 7
    drain(ssem, tile_l, 7, 0)
    drain(ssem, tile_l, 7, 1)

    @pl.when(wid == 0)
    def _():
        pltpu.sync_copy(zbuf_v, out_hbm.at[0, 0, :, 0, :])
        pltpu.sync_copy(zbuf_v, out_hbm.at[0, 1, :, 0, :])
        pltpu.sync_copy(zbuf_v, out_hbm.at[0, 1, :, 1, :])


_mesh = plsc.VectorSubcoreMesh(core_axis_name="c", subcore_axis_name="s",
                               num_cores=NC, num_subcores=NS)

_sc_call = pl.kernel(
    _sc_body,
    out_type=jax.ShapeDtypeStruct((TR, 2, MEM, 8, 2 * MEM), jnp.float32),
    mesh=_mesh,
    scratch_types=[
        pltpu.VMEM((CHUNK,), jnp.int32),
        pltpu.VMEM((P_PER_W, 2, RH), jnp.int32),
        pltpu.VMEM((P_PER_W, 2, RH), jnp.int32),
        pltpu.VMEM((4, 2, 2, RH, MEM), jnp.float32),
        pltpu.VMEM((MEM, 2 * MEM), jnp.float32),
        pltpu.SemaphoreType.DMA,
        pltpu.SemaphoreType.DMA,
    ],
    compiler_params=pltpu.CompilerParams(use_tc_tiling_on_sc=False,
                                         needs_layout_passes=False),
)


@jax.jit
def _run(codes_flat, t2, t3):
    return _sc_call(codes_flat, t2, t3)


def kernel(route_codes_btr, table_ngram_2, table_ngram_3):
    b, s, r = route_codes_btr.shape
    assert (b, s, r) == (1, S, R)
    codes_flat = route_codes_btr.astype(jnp.int32).reshape(-1)
    out5 = _run(codes_flat, table_ngram_2, table_ngram_3)
    return out5.transpose(0, 3, 1, 2, 4).reshape(1, S, 2 * R * MEM)
